# Initial kernel scaffold; baseline (speedup 1.0000x reference)
import jax
import jax.numpy as jnp
from jax.experimental import pallas as pl


def kernel(P, Y, Lam, X, M, W_field, Kset):
    raise NotImplementedError("write your pallas kernel here")


# speedup vs baseline: 1.0802x; 1.0802x over previous
import functools

import jax
import jax.numpy as jnp
from jax import lax
from jax.experimental import pallas as pl
from jax.experimental.pallas import tpu as pltpu
from jax.experimental.pallas import tpu_sc as plsc

_RHO = 0.5
_BETA = float(1.0 + (1.0 / 99.0) * (0.1 - 1.0))

_N, _K, _D = 4096, 32, 64
_B = _N * _K
_NW = 32
_RPW = _B // _NW
_CH = 128
_NCH = _RPW // _CH
_NBUF = 4


def _gather_body(idx_hbm, table_hbm, out_hbm, idx_v, rows_v, in_sems, out_sems):
    cid = lax.axis_index("c")
    sid = lax.axis_index("s")
    wid = sid * 2 + cid

    pltpu.sync_copy(idx_hbm.at[wid], idx_v)

    def g_start(chunk, slot):
        pltpu.make_async_copy(
            table_hbm.at[idx_v.at[chunk]], rows_v.at[slot], in_sems.at[slot]
        ).start()

    def g_wait(slot):
        pltpu.make_async_copy(
            table_hbm.at[idx_v.at[0]], rows_v.at[slot], in_sems.at[slot]
        ).wait()

    def o_start(chunk, slot):
        pltpu.make_async_copy(
            rows_v.at[slot], out_hbm.at[wid, chunk], out_sems.at[slot]
        ).start()

    def o_wait(slot):
        pltpu.make_async_copy(
            rows_v.at[slot], out_hbm.at[wid, 0], out_sems.at[slot]
        ).wait()

    for c in range(_NBUF):
        g_start(c, c)
    for c in range(_NCH):
        b = c % _NBUF
        g_wait(b)
        o_start(c, b)
        nxt = c + _NBUF
        if nxt < _NCH:
            o_wait(b)
            g_start(nxt, b)
    for c in range(_NCH - _NBUF, _NCH):
        o_wait(c % _NBUF)


def _sc_gather(kset_flat, table):
    idx4 = kset_flat.reshape(_NW, _NCH, _CH).astype(jnp.int32)
    mesh = plsc.VectorSubcoreMesh(core_axis_name="c", subcore_axis_name="s")
    fn = pl.kernel(
        _gather_body,
        mesh=mesh,
        out_type=jax.ShapeDtypeStruct((_NW, _NCH, _CH, _D), jnp.float32),
        scratch_types=[
            pltpu.VMEM((_NCH, _CH), jnp.int32),
            pltpu.VMEM((_NBUF, _CH, _D), jnp.float32),
            pltpu.SemaphoreType.DMA((_NBUF,)),
            pltpu.SemaphoreType.DMA((_NBUF,)),
        ],
        compiler_params=pltpu.CompilerParams(use_tc_tiling_on_sc=False),
    )
    return fn(idx4, table).reshape(_B, _D)


_BN = 256
_GRID = _N // _BN


def _dense_body(p_ref, y_ref, lam_ref, x_ref, w_ref, t_ref,
                pn_ref, yn_ref, lamn_ref, en_ref):
    p = p_ref[...]
    t = t_ref[...].reshape(_BN, _K, _D)
    y_new = y_ref[...] + jnp.tanh(
        jnp.dot(x_ref[...], w_ref[...], preferred_element_type=jnp.float32))
    y_from_p = jnp.sum(p[:, :, None] * t, axis=1)
    xi = lam_ref[...] + _RHO * (y_new - y_from_p)
    scores = jnp.sum(t * xi[:, None, :], axis=2)
    logits = jnp.log(p + 1e-9) - _BETA * scores
    m = jnp.max(logits, axis=1, keepdims=True)
    e = jnp.exp(logits - m)
    p_new = e / jnp.sum(e, axis=1, keepdims=True)
    resid2 = y_new - jnp.sum(p_new[:, :, None] * t, axis=1)
    lam_new = lam_ref[...] + _RHO * resid2
    pn_ref[...] = p_new
    yn_ref[...] = y_new
    lamn_ref[...] = lam_new

    @pl.when(pl.program_id(0) == 0)
    def _():
        en_ref[0, 0] = 0.0

    en_ref[0, 0] += (0.5 * _RHO * jnp.sum(resid2 * resid2)
                     + jnp.sum(lam_new * resid2))


def _dense(P, Y, Lam, X, W_field, T):
    out = pl.pallas_call(
        _dense_body,
        grid=(_GRID,),
        in_specs=[
            pl.BlockSpec((_BN, _K), lambda i: (i, 0)),
            pl.BlockSpec((_BN, _D), lambda i: (i, 0)),
            pl.BlockSpec((_BN, _D), lambda i: (i, 0)),
            pl.BlockSpec((_BN, _D), lambda i: (i, 0)),
            pl.BlockSpec((_D, _D), lambda i: (0, 0)),
            pl.BlockSpec((_BN * _K, _D), lambda i: (i, 0)),
        ],
        out_specs=[
            pl.BlockSpec((_BN, _K), lambda i: (i, 0)),
            pl.BlockSpec((_BN, _D), lambda i: (i, 0)),
            pl.BlockSpec((_BN, _D), lambda i: (i, 0)),
            pl.BlockSpec((1, 1), lambda i: (0, 0),
                         memory_space=pltpu.SMEM),
        ],
        out_shape=[
            jax.ShapeDtypeStruct((_N, _K), jnp.float32),
            jax.ShapeDtypeStruct((_N, _D), jnp.float32),
            jax.ShapeDtypeStruct((_N, _D), jnp.float32),
            jax.ShapeDtypeStruct((1, 1), jnp.float32),
        ],
    )(P, Y, Lam, X, W_field, T)
    return out


def kernel(P, Y, Lam, X, M, W_field, Kset):
    T = _sc_gather(Kset.reshape(-1), M)
    p_new, y_new, lam_new, energy = _dense(P, Y, Lam, X, W_field, T)
    return (p_new, y_new, lam_new, energy[0, 0])
